# baseline (speedup 1.0000x reference)
import jax
import jax.numpy as jnp
from jax.experimental import pallas as pl
from jax.experimental.pallas import tpu as pltpu

N, D_IN, D_OUT, E, TOP_K = 4096, 1024, 1024, 8, 2
RT = 2048
TT = 4096
LANES = 128
NEG = -1e30


def _router_body(x_ref, wr_ref, br_ref, wfull_ref):
    lane = jax.lax.broadcasted_iota(jnp.int32, (RT, LANES), 1)
    logits = jnp.dot(x_ref[...], wr_ref[...],
                     preferred_element_type=jnp.float32) + br_ref[...]
    logits = jnp.where(lane < E, logits, NEG)
    m1 = jnp.max(logits, axis=1, keepdims=True)
    i1 = jnp.min(jnp.where(logits == m1, lane, LANES), axis=1, keepdims=True)
    l2 = jnp.where(lane == i1, NEG, logits)
    m2 = jnp.max(l2, axis=1, keepdims=True)
    i2 = jnp.min(jnp.where(l2 == m2, lane, LANES), axis=1, keepdims=True)
    w0 = 1.0 / (1.0 + jnp.exp(m2 - m1))
    w1 = 1.0 - w0
    wfull_ref[...] = jnp.where(lane == i1, w0, 0.0) + jnp.where(
        lane == i2, w1, 0.0)


def _ffn_body(x_ref, wfull_ref, w_ref, b_ref, out_ref):
    e = pl.program_id(1)
    lane = jax.lax.broadcasted_iota(jnp.int32, (TT, LANES), 1)
    w_e = jnp.sum(jnp.where(lane == e, wfull_ref[...], 0.0), axis=1,
                  keepdims=True)
    z = jnp.dot(x_ref[...], w_ref[0],
                preferred_element_type=jnp.float32) + b_ref[0]
    y = w_e * (0.5 * z * (1.0 + jax.lax.erf(z * 0.7071067811865476)))

    @pl.when(e == 0)
    def _init():
        out_ref[...] = y

    @pl.when(e != 0)
    def _acc():
        out_ref[...] += y


@jax.jit
def kernel(x, Wr, br, W, b):
    wr_pad = jnp.zeros((D_IN, LANES), jnp.float32).at[:, :E].set(Wr)
    br_pad = jnp.zeros((1, LANES), jnp.float32).at[0, :E].set(br)
    wfull = pl.pallas_call(
        _router_body,
        grid=(N // RT,),
        in_specs=[
            pl.BlockSpec((RT, D_IN), lambda t: (t, 0)),
            pl.BlockSpec((D_IN, LANES), lambda t: (0, 0)),
            pl.BlockSpec((1, LANES), lambda t: (0, 0)),
        ],
        out_specs=pl.BlockSpec((RT, LANES), lambda t: (t, 0)),
        out_shape=jax.ShapeDtypeStruct((N, LANES), jnp.float32),
    )(x, wr_pad, br_pad)

    x_bf = x.astype(jnp.bfloat16)
    w_bf = W.astype(jnp.bfloat16)
    return pl.pallas_call(
        _ffn_body,
        grid=(N // TT, E),
        in_specs=[
            pl.BlockSpec((TT, D_IN), lambda t, e: (t, 0)),
            pl.BlockSpec((TT, LANES), lambda t, e: (t, 0)),
            pl.BlockSpec((1, D_IN, D_OUT), lambda t, e: (e, 0, 0)),
            pl.BlockSpec((1, 1, D_OUT), lambda t, e: (e, 0, 0)),
        ],
        out_specs=pl.BlockSpec((TT, D_OUT), lambda t, e: (t, 0)),
        out_shape=jax.ShapeDtypeStruct((N, D_OUT), jnp.float32),
        compiler_params=pltpu.CompilerParams(
            dimension_semantics=("arbitrary", "arbitrary"),
        ),
    )(x_bf, wfull, w_bf, b.reshape(E, 1, D_OUT))
